# Initial kernel scaffold; baseline (speedup 1.0000x reference)
import jax
import jax.numpy as jnp
from jax.experimental import pallas as pl


def kernel(x, edge_index, Wp, bp, W0, b0, W1, b1, W2, b2, g0, be0, g1, be1, Aw1, Ab1, Aw2, Ab2, Wc):
    raise NotImplementedError("write your pallas kernel here")


# speedup vs baseline: 12.6851x; 12.6851x over previous
import functools

import jax
import jax.numpy as jnp
from jax import lax
from jax.experimental import pallas as pl
from jax.experimental.pallas import tpu as pltpu
from jax.experimental.pallas import tpu_sc as plsc

N = 10000
D_IN = 128
H = 128
D_OUT = 64
E = 320000

NPAD = 10240
NW = 32
NCH = 79
CB = 128
EPAD = NW * NCH * CB
RPT = NPAD // 16

_MESH = plsc.VectorSubcoreMesh(core_axis_name="c", subcore_axis_name="s")
_BN = float(1.0 / (1.0 + 1e-5) ** 0.5)


def _lrelu(t):
    return jnp.where(t >= 0, t, 0.2 * t)


@functools.partial(
    pl.kernel,
    out_type=jax.ShapeDtypeStruct((2, NPAD), jnp.float32),
    mesh=_MESH,
    scratch_types=[
        pltpu.VMEM((NCH, CB), jnp.int32),
        pltpu.VMEM((CB,), jnp.float32),
        pltpu.VMEM_SHARED((NPAD,), jnp.float32),
    ],
)
def _deg_kernel(dst_hbm, zeros_hbm, out_hbm, dst_v, ones_v, deg_sh):
    cid = lax.axis_index("c")
    sid = lax.axis_index("s")
    wid = cid * 16 + sid
    base = pl.multiple_of(sid * RPT, 8)
    pltpu.sync_copy(zeros_hbm.at[pl.ds(base, RPT)], deg_sh.at[pl.ds(base, RPT)])
    pltpu.sync_copy(dst_hbm.at[wid], dst_v)
    for i in range(CB // 16):
        ones_v[pl.ds(i * 16, 16)] = jnp.ones((16,), jnp.float32)
    plsc.subcore_barrier()

    def body(j, carry):
        pltpu.sync_copy(ones_v, deg_sh.at[dst_v.at[j]], add=True)
        return carry

    lax.fori_loop(0, NCH, body, 0)
    plsc.subcore_barrier()
    pltpu.sync_copy(deg_sh.at[pl.ds(base, RPT)], out_hbm.at[cid, pl.ds(base, RPT)])


def _make_agg(hdim):
    @functools.partial(
        pl.kernel,
        out_type=jax.ShapeDtypeStruct((2, NPAD, hdim), jnp.float32),
        mesh=_MESH,
        scratch_types=[
            pltpu.VMEM((NCH, CB), jnp.int32),
            pltpu.VMEM((NCH, CB), jnp.int32),
            pltpu.VMEM((CB, hdim), jnp.float32),
            pltpu.VMEM_SHARED((NPAD, hdim), jnp.float32),
            pltpu.SemaphoreType.DMA,
        ],
        compiler_params=pltpu.CompilerParams(use_tc_tiling_on_sc=False),
    )
    def _agg(src_hbm, dst_hbm, z_hbm, out_hbm, src_v, dst_v, rows_v, acc_sh, sem):
        cid = lax.axis_index("c")
        sid = lax.axis_index("s")
        wid = cid * 16 + sid
        base = pl.multiple_of(sid * RPT, 8)
        pltpu.sync_copy(z_hbm.at[pl.ds(base, RPT)], acc_sh.at[pl.ds(base, RPT)])
        pltpu.sync_copy(src_hbm.at[wid], src_v)
        pltpu.sync_copy(dst_hbm.at[wid], dst_v)
        plsc.subcore_barrier()

        def body(j, carry):
            pltpu.async_copy(z_hbm.at[src_v.at[j]], rows_v, sem).wait()
            pltpu.sync_copy(rows_v, acc_sh.at[dst_v.at[j]], add=True)
            return carry

        lax.fori_loop(0, NCH, body, 0)
        plsc.subcore_barrier()
        pltpu.sync_copy(acc_sh.at[pl.ds(base, RPT)],
                        out_hbm.at[cid, pl.ds(base, RPT)])

    return _agg


_agg128 = _make_agg(H)
_agg64 = _make_agg(D_OUT)


def _tc_pre_body(x_ref, deg_ref, wp_ref, bp_ref, w0_ref, h0_ref, z0_ref, dinv_ref):
    d = deg_ref[...]
    dinv = lax.rsqrt(d[:, 0:1] + d[:, 1:2] + 1.0)
    h0 = _lrelu(jnp.dot(x_ref[...], wp_ref[...],
                        preferred_element_type=jnp.float32) + bp_ref[...])
    h0_ref[...] = h0
    z0_ref[...] = jnp.dot(h0, w0_ref[...],
                          preferred_element_type=jnp.float32) * dinv
    dinv_ref[...] = dinv


_tc_pre = pl.pallas_call(
    _tc_pre_body,
    out_shape=(
        jax.ShapeDtypeStruct((NPAD, H), jnp.float32),
        jax.ShapeDtypeStruct((NPAD, H), jnp.float32),
        jax.ShapeDtypeStruct((NPAD, 1), jnp.float32),
    ),
)


def _make_tc_mid(dout):
    def body(acc_ref, z_ref, h_ref, dinv_ref, b_ref, g_ref, be_ref, w_ref,
             hn_ref, zn_ref):
        dinv = dinv_ref[...]
        s = (acc_ref[0] + acc_ref[1] - z_ref[...]) * dinv + b_ref[...]
        s = s * g_ref[...] + be_ref[...]
        hn = _lrelu(s) + h_ref[...]
        hn_ref[...] = hn
        zn_ref[...] = jnp.dot(hn, w_ref[...],
                              preferred_element_type=jnp.float32) * dinv

    return pl.pallas_call(
        body,
        out_shape=(
            jax.ShapeDtypeStruct((NPAD, H), jnp.float32),
            jax.ShapeDtypeStruct((NPAD, dout), jnp.float32),
        ),
    )


_tc_mid128 = _make_tc_mid(H)
_tc_mid64 = _make_tc_mid(D_OUT)


def _tc_fin_body(acc_ref, z_ref, dinv_ref, b2_ref, a1_ref, ab1_ref, a2_ref,
                 ab2_ref, wc_ref, out_ref):
    h3p = (acc_ref[0] + acc_ref[1] - z_ref[...]) * dinv_ref[...] + b2_ref[...]
    h3 = h3p[:N]
    a = _lrelu(jnp.dot(h3, a1_ref[...],
                       preferred_element_type=jnp.float32) + ab1_ref[...])
    s = jnp.dot(a, a2_ref[...], preferred_element_type=jnp.float32) + ab2_ref[...]
    m = jnp.max(s, axis=0, keepdims=True)
    e = jnp.exp(s - m)
    sm = e / jnp.sum(e, axis=0, keepdims=True)
    logit = jnp.sum(sm * wc_ref[...], axis=1, keepdims=True)
    cw = 1.0 / (1.0 + jnp.exp(-logit))
    out_ref[...] = h3 * cw


_tc_fin = pl.pallas_call(
    _tc_fin_body,
    out_shape=jax.ShapeDtypeStruct((N, D_OUT), jnp.float32),
)


def kernel(x, edge_index, Wp, bp, W0, b0, W1, b1, W2, b2, g0, be0, g1, be1,
           Aw1, Ab1, Aw2, Ab2, Wc):
    f32 = jnp.float32
    src = edge_index[0].astype(jnp.int32)
    dst = edge_index[1].astype(jnp.int32)
    pad = EPAD - E
    src3 = jnp.concatenate([src, jnp.zeros((pad,), jnp.int32)]).reshape(NW, NCH, CB)
    dst3 = jnp.concatenate([dst, jnp.full((pad,), N, jnp.int32)]).reshape(NW, NCH, CB)
    zerosN = jnp.zeros((NPAD,), f32)
    xpad = jnp.pad(x, ((0, NPAD - N), (0, 0)))

    deg = _deg_kernel(dst3, zerosN)
    degT = deg.T

    h0, z0, dinv = _tc_pre(xpad, degT, Wp, bp.reshape(1, -1), W0)
    acc0 = _agg128(src3, dst3, z0)
    h1, z1 = _tc_mid128(acc0, z0, h0, dinv, b0.reshape(1, -1),
                        (g0 * _BN).reshape(1, -1), be0.reshape(1, -1), W1)
    acc1 = _agg128(src3, dst3, z1)
    h2, z2 = _tc_mid64(acc1, z1, h1, dinv, b1.reshape(1, -1),
                       (g1 * _BN).reshape(1, -1), be1.reshape(1, -1), W2)
    del h2
    acc2 = _agg64(src3, dst3, z2)

    a1cat = jnp.transpose(Aw1, (1, 0, 2)).reshape(D_OUT, D_OUT)
    ab1cat = Ab1.reshape(1, D_OUT)
    a2bd = jax.scipy.linalg.block_diag(*[Aw2[i] for i in range(4)])
    ab2cat = Ab2.reshape(1, 4)
    wcr = Wc.reshape(1, 4)

    return _tc_fin(acc2, z2, dinv, b2.reshape(1, -1), a1cat, ab1cat,
                   a2bd, ab2cat, wcr)
